# baseline (speedup 1.0000x reference)
import jax
import jax.numpy as jnp
from jax import lax
from jax.experimental import pallas as pl
from jax.experimental.pallas import tpu as pltpu

VOCAB = 100000
DIM = 128
BATCH = 1024

_RB = 16
_N_PANELS = BATCH // _RB


def _wr_body(in_ref, out_ref, buf, sem):
    buf[...] = jnp.zeros_like(buf)
    for p in range(_N_PANELS):
        dst = out_ref if p % 2 == 0 else in_ref
        pltpu.make_async_copy(buf, dst.at[pl.ds(p * _RB, _RB), :], sem).start()
    for p in range(_N_PANELS):
        dst = out_ref if p % 2 == 0 else in_ref
        pltpu.make_async_copy(buf, dst.at[pl.ds(p * _RB, _RB), :], sem).wait()


@jax.jit
def _wr_probe(z):
    return pl.pallas_call(
        _wr_body,
        grid=(),
        in_specs=[pl.BlockSpec(memory_space=pl.ANY)],
        out_specs=pl.BlockSpec(memory_space=pl.ANY),
        out_shape=jax.ShapeDtypeStruct((BATCH, VOCAB), jnp.float32),
        input_output_aliases={0: 0},
        scratch_shapes=[
            pltpu.VMEM((_RB, VOCAB), jnp.float32),
            pltpu.SemaphoreType.DMA,
        ],
    )(z)


def kernel(inputs, embed_table, linear_w):
    z = jnp.zeros((BATCH, VOCAB), jnp.float32)
    return _wr_probe(z)


# speedup vs baseline: 1.1619x; 1.1619x over previous
import jax
import jax.numpy as jnp
from jax import lax
from jax.experimental import pallas as pl
from jax.experimental.pallas import tpu as pltpu
from jax.experimental.pallas import tpu_sc as plsc

VOCAB = 100000
DIM = 128
BATCH = 1024

_NC = 2
_NS = 16
_NW = _NC * _NS
_B_PER_W = BATCH // _NW

_V_TILE = 4096


def _gather_body(table_hbm, idx_hbm, out_hbm, idx_v, rows_v, sem):
    wid = lax.axis_index("s") * _NC + lax.axis_index("c")
    base = wid * _B_PER_W
    pltpu.sync_copy(idx_hbm.at[pl.ds(base, _B_PER_W)], idx_v)
    pltpu.async_copy(table_hbm.at[idx_v], rows_v, sem).wait()
    pltpu.sync_copy(rows_v, out_hbm.at[pl.ds(base, _B_PER_W)])


@jax.jit
def _sc_gather(embed_table, idx):
    mesh = plsc.VectorSubcoreMesh(core_axis_name="c", subcore_axis_name="s")
    return pl.kernel(
        _gather_body,
        out_type=jax.ShapeDtypeStruct((BATCH, DIM), jnp.float32),
        mesh=mesh,
        scratch_types=[
            pltpu.VMEM((_B_PER_W,), jnp.int32),
            pltpu.VMEM((_B_PER_W, DIM), jnp.float32),
            pltpu.SemaphoreType.DMA,
        ],
    )(embed_table, idx)


def _mm_body(emb_ref, w_ref, out_ref):
    out_ref[...] = lax.dot_general(
        emb_ref[...], w_ref[...],
        (((1,), (1,)), ((), ())),
        preferred_element_type=jnp.float32,
    )


@jax.jit
def _tc_project(emb, linear_w):
    grid = pl.cdiv(VOCAB, _V_TILE)
    return pl.pallas_call(
        _mm_body,
        grid=(grid,),
        in_specs=[
            pl.BlockSpec((BATCH, DIM), lambda i: (0, 0)),
            pl.BlockSpec((_V_TILE, DIM), lambda i: (i, 0)),
        ],
        out_specs=pl.BlockSpec((BATCH, _V_TILE), lambda i: (0, i)),
        out_shape=jax.ShapeDtypeStruct((BATCH, VOCAB), jnp.float32),
    )(emb, linear_w)


def kernel(inputs, embed_table, linear_w):
    idx = inputs.astype(jnp.int32)
    emb = _sc_gather(embed_table, idx)
    return _tc_project(emb, linear_w)
